# baseline (speedup 1.0000x reference)
import functools

import jax
import jax.numpy as jnp
import numpy as np
from jax.experimental import pallas as pl
from jax.experimental.pallas import tpu as pltpu

DIMS = 3
N_LM = 543
N_COLS = N_LM * DIMS
T = 8192
LIPS = ([61, 185, 40, 39, 37, 0, 267, 269, 270, 409, 291]
        + [146, 91, 181, 84, 17, 314, 405, 321, 375, 291]
        + [78, 191, 80, 81, 82, 13, 312, 311, 310, 415, 308]
        + [78, 95, 88, 178, 87, 14, 317, 402, 318, 324, 308])

HL_LO, HL_HI = 468 * DIMS, 489 * DIMS
POSE_LO, POSE_HI = 489 * DIMS, 522 * DIMS
HR_LO, HR_HI = 522 * DIMS, 543 * DIMS

_lips_cols = (np.asarray(LIPS, np.int32)[:, None] * DIMS
              + np.arange(DIMS, dtype=np.int32)[None, :]).reshape(-1)
_feat_cols = np.concatenate([
    _lips_cols,
    np.arange(HL_LO, HL_HI, dtype=np.int32),
    np.arange(POSE_LO, POSE_HI, dtype=np.int32),
    np.arange(HR_LO, HR_HI, dtype=np.int32),
])
N_FEAT = _feat_cols.shape[0]
_G_np = np.zeros((N_COLS, N_FEAT), np.float32)
_G_np[_feat_cols, np.arange(N_FEAT)] = 1.0

TB = 512


def _stats_kernel(x_ref, g_ref, out_ref, acc_sum, acc_ssq, acc_n):
    i = pl.program_id(0)
    nsteps = pl.num_programs(0)

    @pl.when(i == 0)
    def _init():
        acc_sum[...] = jnp.zeros_like(acc_sum)
        acc_ssq[...] = jnp.zeros_like(acc_ssq)
        acc_n[0] = 0.0
        acc_n[1] = 0.0

    blk = x_ref[...]
    hl = blk[:, HL_LO:HL_HI]
    hr = blk[:, HR_LO:HR_HI]
    hl_bad = jnp.any(jnp.isnan(hl), axis=1, keepdims=True).astype(jnp.float32)
    hr_bad = jnp.any(jnp.isnan(hr), axis=1, keepdims=True).astype(jnp.float32)

    col = jax.lax.broadcasted_iota(jnp.int32, (1, N_COLS), 1)
    is_hl = jnp.logical_and(col >= HL_LO, col < HL_HI).astype(jnp.float32)
    is_hr = (col >= HR_LO).astype(jnp.float32)
    wbad = hl_bad * is_hl + hr_bad * is_hr
    blk2 = jnp.where(wbad == 0.0, blk, 0.0)

    acc_sum[...] += jnp.sum(blk2, axis=0, keepdims=True)
    acc_ssq[...] += jnp.sum(blk2 * blk2, axis=0, keepdims=True)
    acc_n[0] += jnp.float32(TB) - jnp.sum(hl_bad)
    acc_n[1] += jnp.float32(TB) - jnp.sum(hr_bad)

    @pl.when(i == nsteps - 1)
    def _finalize():
        s = acc_sum[...]
        q = acc_ssq[...]
        colv = jax.lax.broadcasted_iota(jnp.int32, (1, N_COLS), 1)
        in_hl = jnp.logical_and(colv >= HL_LO, colv < HL_HI).astype(jnp.float32)
        in_hr = (colv >= HR_LO).astype(jnp.float32)
        n = (jnp.float32(T) + (acc_n[0] - T) * in_hl
             + (acc_n[1] - T) * in_hr)
        mean = s / n
        var = (q - n * mean * mean) / (n - 1.0)
        std = jnp.sqrt(var)
        mean = jnp.where(jnp.isnan(mean), 0.0, mean)
        std = jnp.where(jnp.isnan(std), 0.0, std)
        g = g_ref[...]
        out_ref[0:1, :] = jnp.dot(mean, g, preferred_element_type=jnp.float32,
                                  precision=jax.lax.Precision.HIGHEST)
        out_ref[1:2, :] = jnp.dot(std, g, preferred_element_type=jnp.float32,
                                  precision=jax.lax.Precision.HIGHEST)


@jax.jit
def kernel(x):
    x2 = x.reshape(T, N_COLS)
    g = jnp.asarray(_G_np)
    out = pl.pallas_call(
        _stats_kernel,
        grid=(T // TB,),
        in_specs=[
            pl.BlockSpec((TB, N_COLS), lambda i: (i, 0)),
            pl.BlockSpec((N_COLS, N_FEAT), lambda i: (0, 0)),
        ],
        out_specs=pl.BlockSpec((2, N_FEAT), lambda i: (0, 0)),
        out_shape=jax.ShapeDtypeStruct((2, N_FEAT), jnp.float32),
        scratch_shapes=[
            pltpu.VMEM((1, N_COLS), jnp.float32),
            pltpu.VMEM((1, N_COLS), jnp.float32),
            pltpu.SMEM((2,), jnp.float32),
        ],
    )(x2, g)
    return out.reshape(2 * N_FEAT)


# speedup vs baseline: 4.0950x; 4.0950x over previous
import jax
import jax.numpy as jnp
import numpy as np
from jax.experimental import pallas as pl
from jax.experimental.pallas import tpu as pltpu

DIMS = 3
T = 8192
LIPS = ([61, 185, 40, 39, 37, 0, 267, 269, 270, 409, 291]
        + [146, 91, 181, 84, 17, 314, 405, 321, 375, 291]
        + [78, 191, 80, 81, 82, 13, 312, 311, 310, 415, 308]
        + [78, 95, 88, 178, 87, 14, 317, 402, 318, 324, 308])
N_LIPS = len(LIPS) * DIMS
HL_LO = N_LIPS
HL_HI = HL_LO + 21 * DIMS
POSE_HI = HL_HI + 33 * DIMS
N_FEAT = POSE_HI + 21 * DIMS

TB = 1024


def _stats_kernel(x_ref, out_ref, acc_sum, acc_ssq, acc_n):
    i = pl.program_id(0)
    nsteps = pl.num_programs(0)

    @pl.when(i == 0)
    def _init():
        acc_sum[...] = jnp.zeros_like(acc_sum)
        acc_ssq[...] = jnp.zeros_like(acc_ssq)
        acc_n[0] = 0.0
        acc_n[1] = 0.0

    blk = x_ref[...]
    hl = blk[:, HL_LO:HL_HI]
    hr = blk[:, POSE_HI:N_FEAT]
    hl_bad = jnp.any(jnp.isnan(hl), axis=1, keepdims=True).astype(jnp.float32)
    hr_bad = jnp.any(jnp.isnan(hr), axis=1, keepdims=True).astype(jnp.float32)

    col = jax.lax.broadcasted_iota(jnp.int32, (1, N_FEAT), 1)
    is_hl = jnp.logical_and(col >= HL_LO, col < HL_HI).astype(jnp.float32)
    is_hr = (col >= POSE_HI).astype(jnp.float32)
    wbad = hl_bad * is_hl + hr_bad * is_hr
    blk2 = jnp.where(wbad == 0.0, blk, 0.0)

    acc_sum[...] += jnp.sum(blk2, axis=0, keepdims=True)
    acc_ssq[...] += jnp.sum(blk2 * blk2, axis=0, keepdims=True)
    acc_n[0] += jnp.float32(TB) - jnp.sum(hl_bad)
    acc_n[1] += jnp.float32(TB) - jnp.sum(hr_bad)

    @pl.when(i == nsteps - 1)
    def _finalize():
        s = acc_sum[...]
        q = acc_ssq[...]
        colv = jax.lax.broadcasted_iota(jnp.int32, (1, N_FEAT), 1)
        in_hl = jnp.logical_and(colv >= HL_LO, colv < HL_HI).astype(jnp.float32)
        in_hr = (colv >= POSE_HI).astype(jnp.float32)
        n = (jnp.float32(T) + (acc_n[0] - T) * in_hl
             + (acc_n[1] - T) * in_hr)
        mean = s / n
        var = (q - n * mean * mean) / (n - 1.0)
        std = jnp.sqrt(var)
        out_ref[0:1, :] = jnp.where(jnp.isnan(mean), 0.0, mean)
        out_ref[1:2, :] = jnp.where(jnp.isnan(std), 0.0, std)


@jax.jit
def kernel(x):
    lips_idx = jnp.asarray(np.asarray(LIPS, np.int32))
    xg = jnp.concatenate([
        x[:, lips_idx, :].reshape(T, N_LIPS),
        x[:, 468:543, :].reshape(T, 225),
    ], axis=1)
    out = pl.pallas_call(
        _stats_kernel,
        grid=(T // TB,),
        in_specs=[pl.BlockSpec((TB, N_FEAT), lambda i: (i, 0))],
        out_specs=pl.BlockSpec((2, N_FEAT), lambda i: (0, 0)),
        out_shape=jax.ShapeDtypeStruct((2, N_FEAT), jnp.float32),
        scratch_shapes=[
            pltpu.VMEM((1, N_FEAT), jnp.float32),
            pltpu.VMEM((1, N_FEAT), jnp.float32),
            pltpu.SMEM((2,), jnp.float32),
        ],
    )(xg)
    return out.reshape(2 * N_FEAT)
